# Initial kernel scaffold; baseline (speedup 1.0000x reference)
import jax
import jax.numpy as jnp
from jax.experimental import pallas as pl


def kernel(dense_xyz, dense_feats, nei_inds, dense_xyz_norm, sparse_xyz, sparse_xyz_norm, W_u1, b_u1, W_pe1, b_pe1, W_pe2, b_pe2, W_wn1, b_wn1, W_wn2, b_wn2, W_wn3, b_wn3, W_lin, b_lin, W_u2, b_u2):
    raise NotImplementedError("write your pallas kernel here")


# speedup vs baseline: 1.0692x; 1.0692x over previous
import functools

import jax
import jax.numpy as jnp
from jax import lax
from jax.experimental import pallas as pl
from jax.experimental.pallas import tpu as pltpu
from jax.experimental.pallas import tpu_sc as plsc

N = 50000
M = 25000
K = 16
CIN = 128

P = 256
M_PAD = 25600
NBLK = M_PAD // P
E = K * M_PAD
CH = 512


def _lk(x):
    return jnp.where(x >= 0, x, 0.1 * x)


def _sc_gather(feats, xyz128, idxf):
    info = plsc.get_sparse_core_info()
    nw = info.num_cores * info.num_subcores
    b_per_w = E // nw
    n_chunks = b_per_w // CH
    assert b_per_w % CH == 0 and E % nw == 0
    mesh = plsc.VectorSubcoreMesh(core_axis_name="c", subcore_axis_name="s")

    @functools.partial(
        pl.kernel, mesh=mesh,
        out_type=(
            jax.ShapeDtypeStruct((E, CIN), jnp.float32),
            jax.ShapeDtypeStruct((E, CIN), jnp.float32),
        ),
        scratch_types=[
            pltpu.VMEM((CH,), jnp.int32),
            pltpu.VMEM((CH, CIN), jnp.float32),
            pltpu.SemaphoreType.DMA,
        ],
    )
    def k(feats_hbm, xyz_hbm, idx_hbm, outf_hbm, outx_hbm, idx_v, rows, sem):
        wid = lax.axis_index("s") * info.num_cores + lax.axis_index("c")
        base = wid * b_per_w

        def body(c, carry):
            off = base + c * CH
            pltpu.sync_copy(idx_hbm.at[pl.ds(off, CH)], idx_v)
            pltpu.async_copy(feats_hbm.at[idx_v], rows, sem).wait()
            pltpu.sync_copy(rows, outf_hbm.at[pl.ds(off, CH)])
            pltpu.async_copy(xyz_hbm.at[idx_v], rows, sem).wait()
            pltpu.sync_copy(rows, outx_hbm.at[pl.ds(off, CH)])
            return carry

        lax.fori_loop(0, n_chunks, body, 0)

    return k(feats, xyz128, idxf)


def _compute_body(g2_ref, g3_ref, sx_ref, wu1_ref, bu1_ref,
                  wpe1_ref, bpe1_ref, wpe2_ref, bpe2_ref, wwn1_ref,
                  bwn1_ref, wwn2_ref, bwn2_ref, wwn3_ref, bwn3_ref,
                  wl3_ref, blin_ref, wu2_ref, bu2_ref, out_ref, wni_ref):
    sx = sx_ref[...]

    mx = g2_ref[0]
    for k in range(1, K):
        mx = jnp.maximum(mx, g2_ref[k])

    accw = [None] * 16
    for k in range(K):
        g2k = g2_ref[k]
        gf = jnp.dot(g2k, wu1_ref[...],
                     preferred_element_type=jnp.float32) + bu1_ref[...]
        loc = g3_ref[k][:, 0:16] - sx
        wni_ref[k] = loc
        pe = _lk(jnp.dot(loc, wpe1_ref[...],
                         preferred_element_type=jnp.float32) + bpe1_ref[...])
        pe = _lk(jnp.dot(pe, wpe2_ref[...],
                         preferred_element_type=jnp.float32) + bpe2_ref[...])
        nfk = jnp.concatenate([gf, pe], axis=1)
        w1 = _lk(jnp.dot(loc, wwn1_ref[...],
                         preferred_element_type=jnp.float32) + bwn1_ref[...])
        w2 = jnp.dot(w1, wwn2_ref[...],
                     preferred_element_type=jnp.float32) + bwn2_ref[...]
        w3 = _lk(jnp.dot(w2, wwn3_ref[...],
                         preferred_element_type=jnp.float32) + bwn3_ref[...])
        for w in range(16):
            t = nfk * w3[:, w:w + 1]
            accw[w] = t if k == 0 else accw[w] + t

    acc = None
    for w in range(16):
        m = jnp.dot(accw[w], wl3_ref[w],
                    preferred_element_type=jnp.float32)
        acc = m if acc is None else acc + m
    nf1 = _lk(acc + blin_ref[...])
    nf2 = jnp.dot(nf1, wu2_ref[...],
                  preferred_element_type=jnp.float32) + bu2_ref[...]
    out_ref[...] = _lk(nf2 + mx)


def _compute(g2r, g3r, sx16, wu1, bu1, wpe1p, bpe1, wpe2, bpe2,
             wwn1p, bwn1, wwn2, bwn2, wwn3, bwn3, wl3, blin, wu2, bu2):
    full = lambda a: pl.BlockSpec(a.shape, lambda i: (0,) * a.ndim)
    return pl.pallas_call(
        _compute_body,
        grid=(NBLK,),
        in_specs=[
            pl.BlockSpec((K, P, CIN), lambda i: (0, i, 0)),
            pl.BlockSpec((K, P, CIN), lambda i: (0, i, 0)),
            pl.BlockSpec((P, 16), lambda i: (i, 0)),
            full(wu1), full(bu1),
            full(wpe1p), full(bpe1), full(wpe2), full(bpe2),
            full(wwn1p), full(bwn1), full(wwn2), full(bwn2),
            full(wwn3), full(bwn3), full(wl3), full(blin),
            full(wu2), full(bu2),
        ],
        out_specs=[
            pl.BlockSpec((P, CIN), lambda i: (i, 0)),
            pl.BlockSpec((K, P, 16), lambda i: (0, i, 0)),
        ],
        out_shape=[
            jax.ShapeDtypeStruct((M_PAD, CIN), jnp.float32),
            jax.ShapeDtypeStruct((K, M_PAD, 16), jnp.float32),
        ],
    )(g2r, g3r, sx16, wu1, bu1, wpe1p, bpe1, wpe2, bpe2, wwn1p,
      bwn1, wwn2, bwn2, wwn3, bwn3, wl3, blin, wu2, bu2)


def kernel(dense_xyz, dense_feats, nei_inds, dense_xyz_norm, sparse_xyz,
           sparse_xyz_norm, W_u1, b_u1, W_pe1, b_pe1, W_pe2, b_pe2, W_wn1,
           b_wn1, W_wn2, b_wn2, W_wn3, b_wn3, W_lin, b_lin, W_u2, b_u2):
    df = dense_feats[0]
    xyz128 = jnp.pad(dense_xyz[0], ((0, 0), (0, 125)))

    neiT = jnp.pad(nei_inds[0].T, ((0, 0), (0, M_PAD - M)))
    idxf = neiT.reshape(E)
    gf128, gx128 = _sc_gather(df, xyz128, idxf)

    g2r = gf128.reshape(K, M_PAD, CIN)
    g3r = gx128.reshape(K, M_PAD, CIN)
    sx16 = jnp.pad(sparse_xyz[0], ((0, M_PAD - M), (0, 13)))

    wpe1p = jnp.pad(W_pe1, ((0, 13), (0, 0)))
    wwn1p = jnp.pad(W_wn1, ((0, 13), (0, 0)))
    wl3 = W_lin.reshape(64, 16, 64).transpose(1, 0, 2)

    out_f, wni16 = _compute(
        g2r, g3r, sx16, W_u1, b_u1.reshape(1, -1), wpe1p,
        b_pe1.reshape(1, -1), W_pe2, b_pe2.reshape(1, -1), wwn1p,
        b_wn1.reshape(1, -1), W_wn2, b_wn2.reshape(1, -1), W_wn3,
        b_wn3.reshape(1, -1), wl3, b_lin.reshape(1, -1), W_u2,
        b_u2.reshape(1, -1))

    out = out_f[:M][None]
    wni = wni16.transpose(1, 0, 2)[:M, :, :3][None]
    return out, wni


# speedup vs baseline: 1.2033x; 1.1255x over previous
import functools

import jax
import jax.numpy as jnp
from jax import lax
from jax.experimental import pallas as pl
from jax.experimental.pallas import tpu as pltpu
from jax.experimental.pallas import tpu_sc as plsc

N = 50000
M = 25000
K = 16
CIN = 128

P = 256
M_PAD = 25600
NBLK = M_PAD // P
E = K * M_PAD
CH = 512


def _lk(x):
    return jnp.where(x >= 0, x, 0.1 * x)


def _sc_gather(feats, xyz128, idxf):
    info = plsc.get_sparse_core_info()
    nw = info.num_cores * info.num_subcores
    b_per_w = E // nw
    n_chunks = b_per_w // CH
    assert b_per_w % CH == 0 and E % nw == 0
    mesh = plsc.VectorSubcoreMesh(core_axis_name="c", subcore_axis_name="s")

    @functools.partial(
        pl.kernel, mesh=mesh,
        out_type=(
            jax.ShapeDtypeStruct((E, CIN), jnp.float32),
            jax.ShapeDtypeStruct((E, CIN), jnp.float32),
        ),
        scratch_types=[
            pltpu.VMEM((CH,), jnp.int32),
            pltpu.VMEM((CH, CIN), jnp.float32),
            pltpu.SemaphoreType.DMA,
        ],
    )
    def k(feats_hbm, xyz_hbm, idx_hbm, outf_hbm, outx_hbm, idx_v, rows, sem):
        wid = lax.axis_index("s") * info.num_cores + lax.axis_index("c")
        base = wid * b_per_w

        def body(c, carry):
            off = base + c * CH
            pltpu.sync_copy(idx_hbm.at[pl.ds(off, CH)], idx_v)
            pltpu.async_copy(feats_hbm.at[idx_v], rows, sem).wait()
            pltpu.sync_copy(rows, outf_hbm.at[pl.ds(off, CH)])
            pltpu.async_copy(xyz_hbm.at[idx_v], rows, sem).wait()
            pltpu.sync_copy(rows, outx_hbm.at[pl.ds(off, CH)])
            return carry

        lax.fori_loop(0, n_chunks, body, 0)

    return k(feats, xyz128, idxf)


def _compute_body(g2_ref, g3_ref, sx_ref, wu1_ref, bu1_ref,
                  wpe1_ref, bpe1_ref, wpe2_ref, bpe2_ref, wwn1_ref,
                  bwn1_ref, wwn2_ref, bwn2_ref, wwn3_ref, bwn3_ref,
                  wl3_ref, blin_ref, wu2_ref, bu2_ref, out_ref, wni_ref):
    sx = sx_ref[...]

    mx = g2_ref[0]
    for k in range(1, K):
        mx = jnp.maximum(mx, g2_ref[k])

    g2f = g2_ref[...].reshape(K * P, CIN)
    g3f = g3_ref[...].reshape(K * P, CIN)[:, 0:16]
    sx_all = jnp.broadcast_to(sx[None], (K, P, 16)).reshape(K * P, 16)

    gf = jnp.dot(g2f, wu1_ref[...],
                 preferred_element_type=jnp.float32) + bu1_ref[...]
    loc = g3f - sx_all
    wni_ref[...] = loc.reshape(K, P, 16)
    pe = _lk(jnp.dot(loc, wpe1_ref[...],
                     preferred_element_type=jnp.float32) + bpe1_ref[...])
    pe = _lk(jnp.dot(pe, wpe2_ref[...],
                     preferred_element_type=jnp.float32) + bpe2_ref[...])
    nf_all = jnp.concatenate([gf, pe], axis=1)
    w1 = _lk(jnp.dot(loc, wwn1_ref[...],
                     preferred_element_type=jnp.float32) + bwn1_ref[...])
    w2 = jnp.dot(w1, wwn2_ref[...],
                 preferred_element_type=jnp.float32) + bwn2_ref[...]
    w3_all = _lk(jnp.dot(w2, wwn3_ref[...],
                         preferred_element_type=jnp.float32) + bwn3_ref[...])

    acc = None
    for w in range(16):
        t = (nf_all * w3_all[:, w:w + 1]).reshape(K, P, 64)
        s = t[0]
        for k in range(1, K):
            s = s + t[k]
        m = jnp.dot(s, wl3_ref[w],
                    preferred_element_type=jnp.float32)
        acc = m if acc is None else acc + m
    nf1 = _lk(acc + blin_ref[...])
    nf2 = jnp.dot(nf1, wu2_ref[...],
                  preferred_element_type=jnp.float32) + bu2_ref[...]
    out_ref[...] = _lk(nf2 + mx)


def _compute(g2r, g3r, sx16, wu1, bu1, wpe1p, bpe1, wpe2, bpe2,
             wwn1p, bwn1, wwn2, bwn2, wwn3, bwn3, wl3, blin, wu2, bu2):
    full = lambda a: pl.BlockSpec(a.shape, lambda i: (0,) * a.ndim)
    return pl.pallas_call(
        _compute_body,
        grid=(NBLK,),
        in_specs=[
            pl.BlockSpec((K, P, CIN), lambda i: (0, i, 0)),
            pl.BlockSpec((K, P, CIN), lambda i: (0, i, 0)),
            pl.BlockSpec((P, 16), lambda i: (i, 0)),
            full(wu1), full(bu1),
            full(wpe1p), full(bpe1), full(wpe2), full(bpe2),
            full(wwn1p), full(bwn1), full(wwn2), full(bwn2),
            full(wwn3), full(bwn3), full(wl3), full(blin),
            full(wu2), full(bu2),
        ],
        out_specs=[
            pl.BlockSpec((P, CIN), lambda i: (i, 0)),
            pl.BlockSpec((K, P, 16), lambda i: (0, i, 0)),
        ],
        out_shape=[
            jax.ShapeDtypeStruct((M_PAD, CIN), jnp.float32),
            jax.ShapeDtypeStruct((K, M_PAD, 16), jnp.float32),
        ],
    )(g2r, g3r, sx16, wu1, bu1, wpe1p, bpe1, wpe2, bpe2, wwn1p,
      bwn1, wwn2, bwn2, wwn3, bwn3, wl3, blin, wu2, bu2)


def kernel(dense_xyz, dense_feats, nei_inds, dense_xyz_norm, sparse_xyz,
           sparse_xyz_norm, W_u1, b_u1, W_pe1, b_pe1, W_pe2, b_pe2, W_wn1,
           b_wn1, W_wn2, b_wn2, W_wn3, b_wn3, W_lin, b_lin, W_u2, b_u2):
    df = dense_feats[0]
    xyz128 = jnp.pad(dense_xyz[0], ((0, 0), (0, 125)))

    neiT = jnp.pad(nei_inds[0].T, ((0, 0), (0, M_PAD - M)))
    idxf = neiT.reshape(E)
    gf128, gx128 = _sc_gather(df, xyz128, idxf)

    g2r = gf128.reshape(K, M_PAD, CIN)
    g3r = gx128.reshape(K, M_PAD, CIN)
    sx16 = jnp.pad(sparse_xyz[0], ((0, M_PAD - M), (0, 13)))

    wpe1p = jnp.pad(W_pe1, ((0, 13), (0, 0)))
    wwn1p = jnp.pad(W_wn1, ((0, 13), (0, 0)))
    wl3 = W_lin.reshape(64, 16, 64).transpose(1, 0, 2)

    out_f, wni16 = _compute(
        g2r, g3r, sx16, W_u1, b_u1.reshape(1, -1), wpe1p,
        b_pe1.reshape(1, -1), W_pe2, b_pe2.reshape(1, -1), wwn1p,
        b_wn1.reshape(1, -1), W_wn2, b_wn2.reshape(1, -1), W_wn3,
        b_wn3.reshape(1, -1), wl3, b_lin.reshape(1, -1), W_u2,
        b_u2.reshape(1, -1))

    out = out_f[:M][None]
    wni = wni16.transpose(1, 0, 2)[:M, :, :3][None]
    return out, wni


# speedup vs baseline: 1.3363x; 1.1105x over previous
import functools

import jax
import jax.numpy as jnp
from jax import lax
from jax.experimental import pallas as pl
from jax.experimental.pallas import tpu as pltpu
from jax.experimental.pallas import tpu_sc as plsc

N = 50000
M = 25000
K = 16
CIN = 128

P = 256
M_PAD = 25600
NBLK = M_PAD // P
E = K * M_PAD
CH = 400


def _lk(x):
    return jnp.where(x >= 0, x, 0.1 * x)


def _sc_gather(feats, xyz128, idxf):
    info = plsc.get_sparse_core_info()
    nw = info.num_cores * info.num_subcores
    b_per_w = E // nw
    n_chunks = b_per_w // CH
    assert b_per_w % CH == 0 and E % nw == 0
    mesh = plsc.VectorSubcoreMesh(core_axis_name="c", subcore_axis_name="s")

    @functools.partial(
        pl.kernel, mesh=mesh,
        out_type=(
            jax.ShapeDtypeStruct((E, CIN), jnp.float32),
            jax.ShapeDtypeStruct((E, CIN), jnp.float32),
        ),
        scratch_types=[
            pltpu.VMEM((CH,), jnp.int32),
            pltpu.VMEM((CH, CIN), jnp.float32),
            pltpu.VMEM((CH, CIN), jnp.float32),
            pltpu.SemaphoreType.DMA,
            pltpu.SemaphoreType.DMA,
        ],
    )
    def k(feats_hbm, xyz_hbm, idx_hbm, outf_hbm, outx_hbm, idx_v, rows_f,
          rows_x, semf, semx):
        wid = lax.axis_index("s") * info.num_cores + lax.axis_index("c")
        base = wid * b_per_w

        def body(c, carry):
            off = base + c * CH
            pltpu.sync_copy(idx_hbm.at[pl.ds(off, CH)], idx_v)
            hf = pltpu.async_copy(feats_hbm.at[idx_v], rows_f, semf)
            hx = pltpu.async_copy(xyz_hbm.at[idx_v], rows_x, semx)
            hf.wait()
            pltpu.sync_copy(rows_f, outf_hbm.at[pl.ds(off, CH)])
            hx.wait()
            pltpu.sync_copy(rows_x, outx_hbm.at[pl.ds(off, CH)])
            return carry

        lax.fori_loop(0, n_chunks, body, 0)

    return k(feats, xyz128, idxf)


def _compute_body(g2_ref, g3_ref, sx_ref, wu1_ref, bu1_ref,
                  wpe1_ref, bpe1_ref, wpe2_ref, bpe2_ref, wwn1_ref,
                  bwn1_ref, wwn2_ref, bwn2_ref, wwn3_ref, bwn3_ref,
                  wl3_ref, blin_ref, wu2_ref, bu2_ref, out_ref, wni_ref):
    sx = sx_ref[...]

    mx = g2_ref[0]
    for k in range(1, K):
        mx = jnp.maximum(mx, g2_ref[k])

    g2f = g2_ref[...].reshape(K * P, CIN)
    g3f = g3_ref[...].reshape(K * P, CIN)[:, 0:16]
    sx_all = jnp.broadcast_to(sx[None], (K, P, 16)).reshape(K * P, 16)

    gf = jnp.dot(g2f, wu1_ref[...],
                 preferred_element_type=jnp.float32) + bu1_ref[...]
    loc = g3f - sx_all
    wni_ref[...] = loc.reshape(K, P, 16)
    pe = _lk(jnp.dot(loc, wpe1_ref[...],
                     preferred_element_type=jnp.float32) + bpe1_ref[...])
    pe = _lk(jnp.dot(pe, wpe2_ref[...],
                     preferred_element_type=jnp.float32) + bpe2_ref[...])
    nf_all = jnp.concatenate([gf, pe], axis=1)
    w1 = _lk(jnp.dot(loc, wwn1_ref[...],
                     preferred_element_type=jnp.float32) + bwn1_ref[...])
    w2 = jnp.dot(w1, wwn2_ref[...],
                 preferred_element_type=jnp.float32) + bwn2_ref[...]
    w3_all = _lk(jnp.dot(w2, wwn3_ref[...],
                         preferred_element_type=jnp.float32) + bwn3_ref[...])

    acc = None
    for w in range(16):
        t = (nf_all * w3_all[:, w:w + 1]).reshape(K, P, 64)
        s = t[0]
        for k in range(1, K):
            s = s + t[k]
        m = jnp.dot(s, wl3_ref[w],
                    preferred_element_type=jnp.float32)
        acc = m if acc is None else acc + m
    nf1 = _lk(acc + blin_ref[...])
    nf2 = jnp.dot(nf1, wu2_ref[...],
                  preferred_element_type=jnp.float32) + bu2_ref[...]
    out_ref[...] = _lk(nf2 + mx)


def _compute(g2r, g3r, sx16, wu1, bu1, wpe1p, bpe1, wpe2, bpe2,
             wwn1p, bwn1, wwn2, bwn2, wwn3, bwn3, wl3, blin, wu2, bu2):
    full = lambda a: pl.BlockSpec(a.shape, lambda i: (0,) * a.ndim)
    return pl.pallas_call(
        _compute_body,
        grid=(NBLK,),
        in_specs=[
            pl.BlockSpec((K, P, CIN), lambda i: (0, i, 0)),
            pl.BlockSpec((K, P, CIN), lambda i: (0, i, 0)),
            pl.BlockSpec((P, 16), lambda i: (i, 0)),
            full(wu1), full(bu1),
            full(wpe1p), full(bpe1), full(wpe2), full(bpe2),
            full(wwn1p), full(bwn1), full(wwn2), full(bwn2),
            full(wwn3), full(bwn3), full(wl3), full(blin),
            full(wu2), full(bu2),
        ],
        out_specs=[
            pl.BlockSpec((P, CIN), lambda i: (i, 0)),
            pl.BlockSpec((K, P, 16), lambda i: (0, i, 0)),
        ],
        out_shape=[
            jax.ShapeDtypeStruct((M_PAD, CIN), jnp.float32),
            jax.ShapeDtypeStruct((K, M_PAD, 16), jnp.float32),
        ],
    )(g2r, g3r, sx16, wu1, bu1, wpe1p, bpe1, wpe2, bpe2, wwn1p,
      bwn1, wwn2, bwn2, wwn3, bwn3, wl3, blin, wu2, bu2)


def kernel(dense_xyz, dense_feats, nei_inds, dense_xyz_norm, sparse_xyz,
           sparse_xyz_norm, W_u1, b_u1, W_pe1, b_pe1, W_pe2, b_pe2, W_wn1,
           b_wn1, W_wn2, b_wn2, W_wn3, b_wn3, W_lin, b_lin, W_u2, b_u2):
    df = dense_feats[0]
    xyz128 = jnp.pad(dense_xyz[0], ((0, 0), (0, 125)))

    neiT = jnp.pad(nei_inds[0].T, ((0, 0), (0, M_PAD - M)))
    idxf = neiT.reshape(E)
    gf128, gx128 = _sc_gather(df, xyz128, idxf)

    g2r = gf128.reshape(K, M_PAD, CIN)
    g3r = gx128.reshape(K, M_PAD, CIN)
    sx16 = jnp.pad(sparse_xyz[0], ((0, M_PAD - M), (0, 13)))

    wpe1p = jnp.pad(W_pe1, ((0, 13), (0, 0)))
    wwn1p = jnp.pad(W_wn1, ((0, 13), (0, 0)))
    wl3 = W_lin.reshape(64, 16, 64).transpose(1, 0, 2)

    out_f, wni16 = _compute(
        g2r, g3r, sx16, W_u1, b_u1.reshape(1, -1), wpe1p,
        b_pe1.reshape(1, -1), W_pe2, b_pe2.reshape(1, -1), wwn1p,
        b_wn1.reshape(1, -1), W_wn2, b_wn2.reshape(1, -1), W_wn3,
        b_wn3.reshape(1, -1), wl3, b_lin.reshape(1, -1), W_u2,
        b_u2.reshape(1, -1))

    out = out_f[:M][None]
    wni = wni16.transpose(1, 0, 2)[:M, :, :3][None]
    return out, wni


# speedup vs baseline: 1.3485x; 1.0091x over previous
import functools

import jax
import jax.numpy as jnp
from jax import lax
from jax.experimental import pallas as pl
from jax.experimental.pallas import tpu as pltpu
from jax.experimental.pallas import tpu_sc as plsc

N = 50000
M = 25000
K = 16
CIN = 128

P = 256
M_PAD = 25600
NBLK = M_PAD // P
E = K * M_PAD
CH = 200


def _lk(x):
    return jnp.where(x >= 0, x, 0.1 * x)


def _sc_gather(feats, xyz128, idxf):
    info = plsc.get_sparse_core_info()
    nw = info.num_cores * info.num_subcores
    b_per_w = E // nw
    n_chunks = b_per_w // CH
    assert b_per_w % CH == 0 and E % nw == 0
    mesh = plsc.VectorSubcoreMesh(core_axis_name="c", subcore_axis_name="s")

    @functools.partial(
        pl.kernel, mesh=mesh,
        out_type=(
            jax.ShapeDtypeStruct((E, CIN), jnp.float32),
            jax.ShapeDtypeStruct((E, CIN), jnp.float32),
        ),
        scratch_types=[
            pltpu.VMEM((CH,), jnp.int32),
            pltpu.VMEM((CH,), jnp.int32),
            pltpu.VMEM((CH, CIN), jnp.float32),
            pltpu.VMEM((CH, CIN), jnp.float32),
            pltpu.VMEM((CH, CIN), jnp.float32),
            pltpu.VMEM((CH, CIN), jnp.float32),
            pltpu.SemaphoreType.DMA,
            pltpu.SemaphoreType.DMA,
        ],
    )
    def k(feats_hbm, xyz_hbm, idx_hbm, outf_hbm, outx_hbm, idx_v0, idx_v1,
          rows_f0, rows_f1, rows_x0, rows_x1, semf, semx):
        wid = lax.axis_index("s") * info.num_cores + lax.axis_index("c")
        base = wid * b_per_w

        def body(j, carry):
            off0 = base + (2 * j) * CH
            off1 = off0 + CH
            pltpu.sync_copy(idx_hbm.at[pl.ds(off0, CH)], idx_v0)
            hf0 = pltpu.async_copy(feats_hbm.at[idx_v0], rows_f0, semf)
            hx0 = pltpu.async_copy(xyz_hbm.at[idx_v0], rows_x0, semx)
            pltpu.sync_copy(idx_hbm.at[pl.ds(off1, CH)], idx_v1)
            hf1 = pltpu.async_copy(feats_hbm.at[idx_v1], rows_f1, semf)
            hx1 = pltpu.async_copy(xyz_hbm.at[idx_v1], rows_x1, semx)
            hf0.wait()
            pltpu.sync_copy(rows_f0, outf_hbm.at[pl.ds(off0, CH)])
            hx0.wait()
            pltpu.sync_copy(rows_x0, outx_hbm.at[pl.ds(off0, CH)])
            hf1.wait()
            pltpu.sync_copy(rows_f1, outf_hbm.at[pl.ds(off1, CH)])
            hx1.wait()
            pltpu.sync_copy(rows_x1, outx_hbm.at[pl.ds(off1, CH)])
            return carry

        lax.fori_loop(0, n_chunks // 2, body, 0)

    return k(feats, xyz128, idxf)


def _compute_body(g2_ref, g3_ref, sx_ref, wu1_ref, bu1_ref,
                  wpe1_ref, bpe1_ref, wpe2_ref, bpe2_ref, wwn1_ref,
                  bwn1_ref, wwn2_ref, bwn2_ref, wwn3_ref, bwn3_ref,
                  wl3_ref, blin_ref, wu2_ref, bu2_ref, out_ref, wni_ref):
    sx = sx_ref[...]

    mx = g2_ref[0]
    for k in range(1, K):
        mx = jnp.maximum(mx, g2_ref[k])

    g2f = g2_ref[...].reshape(K * P, CIN)
    g3f = g3_ref[...].reshape(K * P, CIN)[:, 0:16]
    sx_all = jnp.broadcast_to(sx[None], (K, P, 16)).reshape(K * P, 16)

    gf = jnp.dot(g2f, wu1_ref[...],
                 preferred_element_type=jnp.float32) + bu1_ref[...]
    loc = g3f - sx_all
    wni_ref[...] = loc.reshape(K, P, 16)
    pe = _lk(jnp.dot(loc, wpe1_ref[...],
                     preferred_element_type=jnp.float32) + bpe1_ref[...])
    pe = _lk(jnp.dot(pe, wpe2_ref[...],
                     preferred_element_type=jnp.float32) + bpe2_ref[...])
    nf_all = jnp.concatenate([gf, pe], axis=1)
    w1 = _lk(jnp.dot(loc, wwn1_ref[...],
                     preferred_element_type=jnp.float32) + bwn1_ref[...])
    w2 = jnp.dot(w1, wwn2_ref[...],
                 preferred_element_type=jnp.float32) + bwn2_ref[...]
    w3_all = _lk(jnp.dot(w2, wwn3_ref[...],
                         preferred_element_type=jnp.float32) + bwn3_ref[...])

    acc = None
    for w in range(16):
        t = (nf_all * w3_all[:, w:w + 1]).reshape(K, P, 64)
        s = t[0]
        for k in range(1, K):
            s = s + t[k]
        m = jnp.dot(s, wl3_ref[w],
                    preferred_element_type=jnp.float32)
        acc = m if acc is None else acc + m
    nf1 = _lk(acc + blin_ref[...])
    nf2 = jnp.dot(nf1, wu2_ref[...],
                  preferred_element_type=jnp.float32) + bu2_ref[...]
    out_ref[...] = _lk(nf2 + mx)


def _compute(g2r, g3r, sx16, wu1, bu1, wpe1p, bpe1, wpe2, bpe2,
             wwn1p, bwn1, wwn2, bwn2, wwn3, bwn3, wl3, blin, wu2, bu2):
    full = lambda a: pl.BlockSpec(a.shape, lambda i: (0,) * a.ndim)
    return pl.pallas_call(
        _compute_body,
        grid=(NBLK,),
        in_specs=[
            pl.BlockSpec((K, P, CIN), lambda i: (0, i, 0)),
            pl.BlockSpec((K, P, CIN), lambda i: (0, i, 0)),
            pl.BlockSpec((P, 16), lambda i: (i, 0)),
            full(wu1), full(bu1),
            full(wpe1p), full(bpe1), full(wpe2), full(bpe2),
            full(wwn1p), full(bwn1), full(wwn2), full(bwn2),
            full(wwn3), full(bwn3), full(wl3), full(blin),
            full(wu2), full(bu2),
        ],
        out_specs=[
            pl.BlockSpec((P, CIN), lambda i: (i, 0)),
            pl.BlockSpec((K, P, 16), lambda i: (0, i, 0)),
        ],
        out_shape=[
            jax.ShapeDtypeStruct((M_PAD, CIN), jnp.float32),
            jax.ShapeDtypeStruct((K, M_PAD, 16), jnp.float32),
        ],
    )(g2r, g3r, sx16, wu1, bu1, wpe1p, bpe1, wpe2, bpe2, wwn1p,
      bwn1, wwn2, bwn2, wwn3, bwn3, wl3, blin, wu2, bu2)


def kernel(dense_xyz, dense_feats, nei_inds, dense_xyz_norm, sparse_xyz,
           sparse_xyz_norm, W_u1, b_u1, W_pe1, b_pe1, W_pe2, b_pe2, W_wn1,
           b_wn1, W_wn2, b_wn2, W_wn3, b_wn3, W_lin, b_lin, W_u2, b_u2):
    df = dense_feats[0]
    xyz128 = jnp.pad(dense_xyz[0], ((0, 0), (0, 125)))

    neiT = jnp.pad(nei_inds[0].T, ((0, 0), (0, M_PAD - M)))
    idxf = neiT.reshape(E)
    gf128, gx128 = _sc_gather(df, xyz128, idxf)

    g2r = gf128.reshape(K, M_PAD, CIN)
    g3r = gx128.reshape(K, M_PAD, CIN)
    sx16 = jnp.pad(sparse_xyz[0], ((0, M_PAD - M), (0, 13)))

    wpe1p = jnp.pad(W_pe1, ((0, 13), (0, 0)))
    wwn1p = jnp.pad(W_wn1, ((0, 13), (0, 0)))
    wl3 = W_lin.reshape(64, 16, 64).transpose(1, 0, 2)

    out_f, wni16 = _compute(
        g2r, g3r, sx16, W_u1, b_u1.reshape(1, -1), wpe1p,
        b_pe1.reshape(1, -1), W_pe2, b_pe2.reshape(1, -1), wwn1p,
        b_wn1.reshape(1, -1), W_wn2, b_wn2.reshape(1, -1), W_wn3,
        b_wn3.reshape(1, -1), wl3, b_lin.reshape(1, -1), W_u2,
        b_u2.reshape(1, -1))

    out = out_f[:M][None]
    wni = wni16.transpose(1, 0, 2)[:M, :, :3][None]
    return out, wni
